# Initial kernel scaffold; baseline (speedup 1.0000x reference)
import functools

import jax
import jax.numpy as jnp
from jax import lax
from jax.experimental import pallas as pl
from jax.experimental.pallas import tpu as pltpu
from jax.experimental.pallas import tpu_sc as plsc

N = 10000
E = 320000
D = 128

NC = 2
NS = 16
K = 80
NCHUNK = E // (NS * K)
ROWS_PER_TILE = N // NS
NPAD = 10240
HSLICE = NPAD // NS

_mesh = plsc.VectorSubcoreMesh(core_axis_name="c", subcore_axis_name="s")


def _degrees_body(e0_hbm, e1_hbm, out0_hbm, out1_hbm,
                  hist_s, hist_d, idx_s, idx_d, shared):
    c = lax.axis_index("c")
    s = lax.axis_index("s")
    zero16 = jnp.zeros((16,), jnp.float32)

    def zero_body(i, _):
        hist_s[pl.ds(i * 16, 16)] = zero16
        hist_d[pl.ds(i * 16, 16)] = zero16
        return 0

    lax.fori_loop(0, NPAD // 16, zero_body, 0)

    @pl.when(s < 2)
    def _():
        pltpu.sync_copy(hist_s, shared.at[s])

    @pl.when(c == 0)
    def _():
        pltpu.sync_copy(e0_hbm.at[0, s], idx_s)
        pltpu.sync_copy(e0_hbm.at[1, s], idx_d)

    @pl.when(c == 1)
    def _():
        pltpu.sync_copy(e1_hbm.at[0, s], idx_s)
        pltpu.sync_copy(e1_hbm.at[1, s], idx_d)

    plsc.subcore_barrier()

    one16 = jnp.full((16,), 1.0, jnp.float32)

    def acc_body(r, _):
        for j in range(K // 16):
            plsc.addupdate_scatter(hist_s, [idx_s[r, pl.ds(j * 16, 16)]], one16)
            plsc.addupdate_scatter(hist_d, [idx_d[r, pl.ds(j * 16, 16)]], one16)
        return 0

    lax.fori_loop(0, NCHUNK, acc_body, 0)

    pltpu.sync_copy(hist_s, shared.at[0], add=True)
    pltpu.sync_copy(hist_d, shared.at[1], add=True)
    plsc.subcore_barrier()

    @pl.when(c == 0)
    def _():
        for j in range(2):
            pltpu.sync_copy(shared.at[j, pl.ds(s * HSLICE, HSLICE)],
                            out0_hbm.at[j, pl.ds(s * HSLICE, HSLICE)])

    @pl.when(c == 1)
    def _():
        for j in range(2):
            pltpu.sync_copy(shared.at[j, pl.ds(s * HSLICE, HSLICE)],
                            out1_hbm.at[j, pl.ds(s * HSLICE, HSLICE)])


_sc_degrees = pl.kernel(
    _degrees_body,
    out_type=(jax.ShapeDtypeStruct((2, NPAD), jnp.float32),
              jax.ShapeDtypeStruct((2, NPAD), jnp.float32)),
    mesh=_mesh,
    scratch_types=[
        pltpu.VMEM((NPAD,), jnp.float32),
        pltpu.VMEM((NPAD,), jnp.float32),
        pltpu.VMEM((NCHUNK, K), jnp.int32),
        pltpu.VMEM((NCHUNK, K), jnp.int32),
        pltpu.VMEM_SHARED((2, NPAD), jnp.float32),
    ],
)


def _matmul_body(x_ref, w0_ref, w1_ref, d0_ref, d1_ref, t0_ref, t1_ref):
    x = x_ref[...]
    s0 = lax.rsqrt(jnp.maximum(d0_ref[...], 1.0))
    s1 = lax.rsqrt(jnp.maximum(d1_ref[...], 1.0))
    t0_ref[...] = jnp.dot(x * s0, w0_ref[...], preferred_element_type=jnp.float32)
    t1_ref[...] = jnp.dot(x * s1, w1_ref[...], preferred_element_type=jnp.float32)


def _tc_matmul(x, w0, w1, d0, d1):
    blk = 1000
    grid = (N // blk,)
    return pl.pallas_call(
        _matmul_body,
        grid=grid,
        in_specs=[
            pl.BlockSpec((blk, D), lambda i: (i, 0)),
            pl.BlockSpec((D, D), lambda i: (0, 0)),
            pl.BlockSpec((D, D), lambda i: (0, 0)),
            pl.BlockSpec((blk, 1), lambda i: (i, 0)),
            pl.BlockSpec((blk, 1), lambda i: (i, 0)),
        ],
        out_specs=[
            pl.BlockSpec((blk, D), lambda i: (i, 0)),
            pl.BlockSpec((blk, D), lambda i: (i, 0)),
        ],
        out_shape=[
            jax.ShapeDtypeStruct((N, D), jnp.float32),
            jax.ShapeDtypeStruct((N, D), jnp.float32),
        ],
    )(x, w0, w1, d0, d1)


def _scatter_body(t0_hbm, t1_hbm, e0_hbm, e1_hbm, z_hbm, agg0_hbm, agg1_hbm,
                  idx_s, idx_d, rows, agg_sh, sem_a, sem_b):
    c = lax.axis_index("c")
    s = lax.axis_index("s")

    pltpu.sync_copy(z_hbm, agg_sh.at[pl.ds(s * ROWS_PER_TILE, ROWS_PER_TILE)])

    @pl.when(c == 0)
    def _():
        pltpu.sync_copy(e0_hbm.at[0, s], idx_s)
        pltpu.sync_copy(e0_hbm.at[1, s], idx_d)

    @pl.when(c == 1)
    def _():
        pltpu.sync_copy(e1_hbm.at[0, s], idx_s)
        pltpu.sync_copy(e1_hbm.at[1, s], idx_d)

    plsc.subcore_barrier()

    def run(t_hbm):
        def gather(k, b, sem):
            return pltpu.make_async_copy(t_hbm.at[idx_s.at[k]], rows.at[b], sem)

        gather(0, 0, sem_a).start()

        def body(i, _):
            k = i * 2
            gather(k, 0, sem_a).wait()
            gather(k + 1, 1, sem_b).start()
            pltpu.sync_copy(rows.at[0], agg_sh.at[idx_d.at[k]], add=True)
            gather(k + 1, 1, sem_b).wait()

            @pl.when(k + 2 < NCHUNK)
            def _():
                gather(k + 2, 0, sem_a).start()

            pltpu.sync_copy(rows.at[1], agg_sh.at[idx_d.at[k + 1]], add=True)
            return 0

        lax.fori_loop(0, NCHUNK // 2, body, 0)

    @pl.when(c == 0)
    def _():
        run(t0_hbm)

    @pl.when(c == 1)
    def _():
        run(t1_hbm)

    plsc.subcore_barrier()

    row_slice = pl.ds(s * ROWS_PER_TILE, ROWS_PER_TILE)

    @pl.when(c == 0)
    def _():
        pltpu.sync_copy(agg_sh.at[row_slice], agg0_hbm.at[row_slice])

    @pl.when(c == 1)
    def _():
        pltpu.sync_copy(agg_sh.at[row_slice], agg1_hbm.at[row_slice])


_sc_scatter = pl.kernel(
    _scatter_body,
    out_type=(jax.ShapeDtypeStruct((N, D), jnp.float32),
              jax.ShapeDtypeStruct((N, D), jnp.float32)),
    mesh=_mesh,
    scratch_types=[
        pltpu.VMEM((NCHUNK, K), jnp.int32),
        pltpu.VMEM((NCHUNK, K), jnp.int32),
        pltpu.VMEM((2, K, D), jnp.float32),
        pltpu.VMEM_SHARED((N, D), jnp.float32),
        pltpu.SemaphoreType.DMA,
        pltpu.SemaphoreType.DMA,
    ],
)


def _combine_body(a0_ref, a1_ref, d0_ref, d1_ref, b0_ref, b1_ref, out_ref):
    s0 = lax.rsqrt(jnp.maximum(d0_ref[...], 1.0))
    s1 = lax.rsqrt(jnp.maximum(d1_ref[...], 1.0))
    h = a0_ref[...] * s0 + a1_ref[...] * s1 + b0_ref[...] + b1_ref[...]
    ss = jnp.sum(h * h, axis=1, keepdims=True)
    out_ref[...] = h * lax.rsqrt(jnp.maximum(ss, 1e-24))


def _tc_combine(a0, a1, d0, d1, b0, b1):
    blk = 1000
    grid = (N // blk,)
    return pl.pallas_call(
        _combine_body,
        grid=grid,
        in_specs=[
            pl.BlockSpec((blk, D), lambda i: (i, 0)),
            pl.BlockSpec((blk, D), lambda i: (i, 0)),
            pl.BlockSpec((blk, 1), lambda i: (i, 0)),
            pl.BlockSpec((blk, 1), lambda i: (i, 0)),
            pl.BlockSpec((1, D), lambda i: (0, 0)),
            pl.BlockSpec((1, D), lambda i: (0, 0)),
        ],
        out_specs=pl.BlockSpec((blk, D), lambda i: (i, 0)),
        out_shape=jax.ShapeDtypeStruct((N, D), jnp.float32),
    )(a0, a1, d0, d1, b0, b1)


@jax.jit
def kernel(x, edge_index_r0, edge_index_r1, W_r0, b_r0, W_r1, b_r1):
    e0 = edge_index_r0.reshape(2, NS, NCHUNK, K)
    e1 = edge_index_r1.reshape(2, NS, NCHUNK, K)

    deg0, deg1 = _sc_degrees(e0, e1)
    dout0 = deg0[0, :N, None]
    din0 = deg0[1, :N, None]
    dout1 = deg1[0, :N, None]
    din1 = deg1[1, :N, None]

    t0, t1 = _tc_matmul(x, W_r0, W_r1, dout0, dout1)

    zeros = jnp.zeros((ROWS_PER_TILE, D), jnp.float32)
    agg0, agg1 = _sc_scatter(t0, t1, e0, e1, zeros)

    return _tc_combine(agg0, agg1, din0, din1, b_r0[None, :], b_r1[None, :])


# speedup vs baseline: 7.3680x; 7.3680x over previous
import functools

import jax
import jax.numpy as jnp
from jax import lax
from jax.experimental import pallas as pl
from jax.experimental.pallas import tpu as pltpu
from jax.experimental.pallas import tpu_sc as plsc

N = 10000
E = 320000
D = 128
DH = D // 2

NC = 2
NS = 16
K = 80
NCHUNK = E // (NS * K)
NPAD = 10240
RPT = NPAD // NS
HSLICE = NPAD // NS

_mesh = plsc.VectorSubcoreMesh(core_axis_name="c", subcore_axis_name="s")


def _degrees_body(e0_hbm, e1_hbm, out0_hbm, out1_hbm,
                  hist_s, hist_d, idx_s, idx_d, tmp, acc_s, acc_d, sh_s, sh_d):
    c = lax.axis_index("c")
    s = lax.axis_index("s")
    zero16 = jnp.zeros((16,), jnp.float32)

    def zero_body(i, _):
        hist_s[pl.ds(i * 16, 16)] = zero16
        hist_d[pl.ds(i * 16, 16)] = zero16
        return 0

    lax.fori_loop(0, NPAD // 16, zero_body, 0)

    @pl.when(c == 0)
    def _():
        pltpu.sync_copy(e0_hbm.at[0, s], idx_s)
        pltpu.sync_copy(e0_hbm.at[1, s], idx_d)

    @pl.when(c == 1)
    def _():
        pltpu.sync_copy(e1_hbm.at[0, s], idx_s)
        pltpu.sync_copy(e1_hbm.at[1, s], idx_d)

    one16 = jnp.full((16,), 1.0, jnp.float32)

    def acc_body(r, _):
        for j in range(K // 16):
            plsc.addupdate_scatter(hist_s, [idx_s[r, pl.ds(j * 16, 16)]], one16)
            plsc.addupdate_scatter(hist_d, [idx_d[r, pl.ds(j * 16, 16)]], one16)
        return 0

    lax.fori_loop(0, NCHUNK, acc_body, 0)

    pltpu.sync_copy(hist_s, sh_s.at[s])
    pltpu.sync_copy(hist_d, sh_d.at[s])
    plsc.subcore_barrier()

    col = pl.ds(s * HSLICE, HSLICE)

    def reduce(sh, acc):
        for t in range(NS):
            pltpu.sync_copy(sh.at[t, col], tmp.at[t])

        def red_body(j, _):
            v = tmp[0, pl.ds(j * 16, 16)]
            for t in range(1, NS):
                v = v + tmp[t, pl.ds(j * 16, 16)]
            acc[pl.ds(j * 16, 16)] = v
            return 0

        lax.fori_loop(0, HSLICE // 16, red_body, 0)

    reduce(sh_s, acc_s)
    reduce(sh_d, acc_d)

    @pl.when(c == 0)
    def _():
        pltpu.sync_copy(acc_s, out0_hbm.at[0, col])
        pltpu.sync_copy(acc_d, out0_hbm.at[1, col])

    @pl.when(c == 1)
    def _():
        pltpu.sync_copy(acc_s, out1_hbm.at[0, col])
        pltpu.sync_copy(acc_d, out1_hbm.at[1, col])


_sc_degrees = pl.kernel(
    _degrees_body,
    out_type=(jax.ShapeDtypeStruct((2, NPAD), jnp.float32),
              jax.ShapeDtypeStruct((2, NPAD), jnp.float32)),
    mesh=_mesh,
    scratch_types=[
        pltpu.VMEM((NPAD,), jnp.float32),
        pltpu.VMEM((NPAD,), jnp.float32),
        pltpu.VMEM((NCHUNK, K), jnp.int32),
        pltpu.VMEM((NCHUNK, K), jnp.int32),
        pltpu.VMEM((NS, HSLICE), jnp.float32),
        pltpu.VMEM((HSLICE,), jnp.float32),
        pltpu.VMEM((HSLICE,), jnp.float32),
        pltpu.VMEM_SHARED((NS, NPAD), jnp.float32),
        pltpu.VMEM_SHARED((NS, NPAD), jnp.float32),
    ],
    compiler_params=pltpu.CompilerParams(needs_layout_passes=False),
)


def _matmul_body(x_ref, w0_ref, w1_ref, d0_ref, d1_ref,
                 t0a_ref, t0b_ref, t1a_ref, t1b_ref):
    x = x_ref[...]
    s0 = lax.rsqrt(jnp.maximum(d0_ref[...], 1.0))
    s1 = lax.rsqrt(jnp.maximum(d1_ref[...], 1.0))
    t0 = jnp.dot(x * s0, w0_ref[...], preferred_element_type=jnp.float32)
    t1 = jnp.dot(x * s1, w1_ref[...], preferred_element_type=jnp.float32)
    t0a_ref[...] = t0[:, :DH]
    t0b_ref[...] = t0[:, DH:]
    t1a_ref[...] = t1[:, :DH]
    t1b_ref[...] = t1[:, DH:]


def _tc_matmul(x, w0, w1, d0, d1):
    blk = 1000
    grid = (N // blk,)
    half = pl.BlockSpec((blk, DH), lambda i: (i, 0))
    return pl.pallas_call(
        _matmul_body,
        grid=grid,
        in_specs=[
            pl.BlockSpec((blk, D), lambda i: (i, 0)),
            pl.BlockSpec((D, D), lambda i: (0, 0)),
            pl.BlockSpec((D, D), lambda i: (0, 0)),
            pl.BlockSpec((blk, 1), lambda i: (i, 0)),
            pl.BlockSpec((blk, 1), lambda i: (i, 0)),
        ],
        out_specs=[half, half, half, half],
        out_shape=[jax.ShapeDtypeStruct((N, DH), jnp.float32)] * 4,
    )(x, w0, w1, d0, d1)


def _scatter_body(t0_hbm, t1_hbm, e0_hbm, e1_hbm, z_hbm, agg0_hbm, agg1_hbm,
                  idx_s, idx_d, rows, agg_sh, sem_a, sem_b):
    c = lax.axis_index("c")
    s = lax.axis_index("s")

    pltpu.sync_copy(z_hbm, agg_sh.at[pl.ds(s * RPT, RPT)])

    @pl.when(c == 0)
    def _():
        pltpu.sync_copy(e0_hbm.at[0, s], idx_s)
        pltpu.sync_copy(e0_hbm.at[1, s], idx_d)

    @pl.when(c == 1)
    def _():
        pltpu.sync_copy(e1_hbm.at[0, s], idx_s)
        pltpu.sync_copy(e1_hbm.at[1, s], idx_d)

    plsc.subcore_barrier()

    def run(t_hbm):
        def gather(k, b, sem):
            return pltpu.make_async_copy(t_hbm.at[idx_s.at[k]], rows.at[b], sem)

        gather(0, 0, sem_a).start()

        def body(i, _):
            k = i * 2
            gather(k, 0, sem_a).wait()
            gather(k + 1, 1, sem_b).start()
            pltpu.sync_copy(rows.at[0], agg_sh.at[idx_d.at[k]], add=True)
            gather(k + 1, 1, sem_b).wait()

            @pl.when(k + 2 < NCHUNK)
            def _():
                gather(k + 2, 0, sem_a).start()

            pltpu.sync_copy(rows.at[1], agg_sh.at[idx_d.at[k + 1]], add=True)
            return 0

        lax.fori_loop(0, NCHUNK // 2, body, 0)

    @pl.when(c == 0)
    def _():
        run(t0_hbm)

    @pl.when(c == 1)
    def _():
        run(t1_hbm)

    plsc.subcore_barrier()

    row_slice = pl.ds(s * RPT, RPT)

    @pl.when(c == 0)
    def _():
        pltpu.sync_copy(agg_sh.at[row_slice], agg0_hbm.at[row_slice])

    @pl.when(c == 1)
    def _():
        pltpu.sync_copy(agg_sh.at[row_slice], agg1_hbm.at[row_slice])


_sc_scatter = pl.kernel(
    _scatter_body,
    out_type=(jax.ShapeDtypeStruct((NPAD, DH), jnp.float32),
              jax.ShapeDtypeStruct((NPAD, DH), jnp.float32)),
    mesh=_mesh,
    scratch_types=[
        pltpu.VMEM((NCHUNK, K), jnp.int32),
        pltpu.VMEM((NCHUNK, K), jnp.int32),
        pltpu.VMEM((2, K, DH), jnp.float32),
        pltpu.VMEM_SHARED((NPAD, DH), jnp.float32),
        pltpu.SemaphoreType.DMA,
        pltpu.SemaphoreType.DMA,
    ],
    compiler_params=pltpu.CompilerParams(use_tc_tiling_on_sc=False),
)


def _combine_body(a0a_ref, a0b_ref, a1a_ref, a1b_ref, d0_ref, d1_ref,
                  b0_ref, b1_ref, out_ref):
    s0 = lax.rsqrt(jnp.maximum(d0_ref[...], 1.0))
    s1 = lax.rsqrt(jnp.maximum(d1_ref[...], 1.0))
    b = b0_ref[...] + b1_ref[...]
    ha = a0a_ref[...] * s0 + a1a_ref[...] * s1 + b[:, :DH]
    hb = a0b_ref[...] * s0 + a1b_ref[...] * s1 + b[:, DH:]
    ss = (jnp.sum(ha * ha, axis=1, keepdims=True)
          + jnp.sum(hb * hb, axis=1, keepdims=True))
    inv = lax.rsqrt(jnp.maximum(ss, 1e-24))
    out_ref[:, :DH] = ha * inv
    out_ref[:, DH:] = hb * inv


def _tc_combine(a0a, a0b, a1a, a1b, d0, d1, b0, b1):
    blk = 1000
    grid = (N // blk,)
    half = pl.BlockSpec((blk, DH), lambda i: (i, 0))
    return pl.pallas_call(
        _combine_body,
        grid=grid,
        in_specs=[
            half, half, half, half,
            pl.BlockSpec((blk, 1), lambda i: (i, 0)),
            pl.BlockSpec((blk, 1), lambda i: (i, 0)),
            pl.BlockSpec((1, D), lambda i: (0, 0)),
            pl.BlockSpec((1, D), lambda i: (0, 0)),
        ],
        out_specs=pl.BlockSpec((blk, D), lambda i: (i, 0)),
        out_shape=jax.ShapeDtypeStruct((N, D), jnp.float32),
    )(a0a, a0b, a1a, a1b, d0, d1, b0, b1)


@jax.jit
def kernel(x, edge_index_r0, edge_index_r1, W_r0, b_r0, W_r1, b_r1):
    e0 = edge_index_r0.reshape(2, NS, NCHUNK, K)
    e1 = edge_index_r1.reshape(2, NS, NCHUNK, K)

    deg0, deg1 = _sc_degrees(e0, e1)
    dout0 = deg0[0, :N, None]
    din0 = deg0[1, :N, None]
    dout1 = deg1[0, :N, None]
    din1 = deg1[1, :N, None]

    t0a, t0b, t1a, t1b = _tc_matmul(x, W_r0, W_r1, dout0, dout1)

    zeros = jnp.zeros((RPT, DH), jnp.float32)
    a0a, a1a = _sc_scatter(t0a, t1a, e0, e1, zeros)
    a0b, a1b = _sc_scatter(t0b, t1b, e0, e1, zeros)

    return _tc_combine(a0a[:N], a0b[:N], a1a[:N], a1b[:N], din0, din1,
                       b_r0[None, :], b_r1[None, :])


# speedup vs baseline: 13.4625x; 1.8271x over previous
import functools

import jax
import jax.numpy as jnp
from jax import lax
from jax.experimental import pallas as pl
from jax.experimental.pallas import tpu as pltpu
from jax.experimental.pallas import tpu_sc as plsc

N = 10000
E = 320000
D = 128
DH = D // 2

NC = 2
NS = 16
K = 80
NCHUNK = E // (NS * K)
KC = 125
NCHUNKC = E // (NS * KC)
NBUF = 4
NPAD = 10240
RPT = NPAD // NS
HSLICE = NPAD // NS

_mesh = plsc.VectorSubcoreMesh(core_axis_name="c", subcore_axis_name="s")


def _degrees_body(e0_hbm, e1_hbm, out0_hbm, out1_hbm,
                  hist_s, hist_d, idx_s, idx_d, tmp, acc_s, acc_d, sh_s, sh_d):
    c = lax.axis_index("c")
    s = lax.axis_index("s")
    zero16 = jnp.zeros((16,), jnp.float32)

    def zero_body(i, _):
        hist_s[pl.ds(i * 16, 16)] = zero16
        hist_d[pl.ds(i * 16, 16)] = zero16
        return 0

    lax.fori_loop(0, NPAD // 16, zero_body, 0)

    @pl.when(c == 0)
    def _():
        pltpu.sync_copy(e0_hbm.at[0, s], idx_s)
        pltpu.sync_copy(e0_hbm.at[1, s], idx_d)

    @pl.when(c == 1)
    def _():
        pltpu.sync_copy(e1_hbm.at[0, s], idx_s)
        pltpu.sync_copy(e1_hbm.at[1, s], idx_d)

    one16 = jnp.full((16,), 1.0, jnp.float32)

    def acc_body(r, _):
        for j in range(K // 16):
            plsc.addupdate_scatter(hist_s, [idx_s[r, pl.ds(j * 16, 16)]], one16)
            plsc.addupdate_scatter(hist_d, [idx_d[r, pl.ds(j * 16, 16)]], one16)
        return 0

    lax.fori_loop(0, NCHUNK, acc_body, 0)

    pltpu.sync_copy(hist_s, sh_s.at[s])
    pltpu.sync_copy(hist_d, sh_d.at[s])
    plsc.subcore_barrier()

    col = pl.ds(s * HSLICE, HSLICE)

    def reduce(sh, acc):
        for t in range(NS):
            pltpu.sync_copy(sh.at[t, col], tmp.at[t])

        def red_body(j, _):
            v = tmp[0, pl.ds(j * 16, 16)]
            for t in range(1, NS):
                v = v + tmp[t, pl.ds(j * 16, 16)]
            acc[pl.ds(j * 16, 16)] = v
            return 0

        lax.fori_loop(0, HSLICE // 16, red_body, 0)

    reduce(sh_s, acc_s)
    reduce(sh_d, acc_d)

    @pl.when(c == 0)
    def _():
        pltpu.sync_copy(acc_s, out0_hbm.at[0, col])
        pltpu.sync_copy(acc_d, out0_hbm.at[1, col])

    @pl.when(c == 1)
    def _():
        pltpu.sync_copy(acc_s, out1_hbm.at[0, col])
        pltpu.sync_copy(acc_d, out1_hbm.at[1, col])


_sc_degrees = pl.kernel(
    _degrees_body,
    out_type=(jax.ShapeDtypeStruct((2, NPAD), jnp.float32),
              jax.ShapeDtypeStruct((2, NPAD), jnp.float32)),
    mesh=_mesh,
    scratch_types=[
        pltpu.VMEM((NPAD,), jnp.float32),
        pltpu.VMEM((NPAD,), jnp.float32),
        pltpu.VMEM((NCHUNK, K), jnp.int32),
        pltpu.VMEM((NCHUNK, K), jnp.int32),
        pltpu.VMEM((NS, HSLICE), jnp.float32),
        pltpu.VMEM((HSLICE,), jnp.float32),
        pltpu.VMEM((HSLICE,), jnp.float32),
        pltpu.VMEM_SHARED((NS, NPAD), jnp.float32),
        pltpu.VMEM_SHARED((NS, NPAD), jnp.float32),
    ],
    compiler_params=pltpu.CompilerParams(needs_layout_passes=False),
)


def _matmul_body(x_ref, w0_ref, w1_ref, d0_ref, d1_ref,
                 t0a_ref, t0b_ref, t1a_ref, t1b_ref):
    x = x_ref[...]
    s0 = lax.rsqrt(jnp.maximum(d0_ref[...], 1.0))
    s1 = lax.rsqrt(jnp.maximum(d1_ref[...], 1.0))
    t0 = jnp.dot(x * s0, w0_ref[...], preferred_element_type=jnp.float32)
    t1 = jnp.dot(x * s1, w1_ref[...], preferred_element_type=jnp.float32)
    t0a_ref[...] = t0[:, :DH]
    t0b_ref[...] = t0[:, DH:]
    t1a_ref[...] = t1[:, :DH]
    t1b_ref[...] = t1[:, DH:]


def _tc_matmul(x, w0, w1, d0, d1):
    blk = 1000
    grid = (N // blk,)
    half = pl.BlockSpec((blk, DH), lambda i: (i, 0))
    return pl.pallas_call(
        _matmul_body,
        grid=grid,
        in_specs=[
            pl.BlockSpec((blk, D), lambda i: (i, 0)),
            pl.BlockSpec((D, D), lambda i: (0, 0)),
            pl.BlockSpec((D, D), lambda i: (0, 0)),
            pl.BlockSpec((blk, 1), lambda i: (i, 0)),
            pl.BlockSpec((blk, 1), lambda i: (i, 0)),
        ],
        out_specs=[half, half, half, half],
        out_shape=[jax.ShapeDtypeStruct((N, DH), jnp.float32)] * 4,
    )(x, w0, w1, d0, d1)


def _scatter_body(t0_hbm, t1_hbm, e0_hbm, e1_hbm, z_hbm, agg0_hbm, agg1_hbm,
                  idx_s, idx_d, rows, agg_sh, sem0, sem1, sem2, sem3):
    c = lax.axis_index("c")
    s = lax.axis_index("s")
    sems = (sem0, sem1, sem2, sem3)

    pltpu.sync_copy(z_hbm, agg_sh.at[pl.ds(s * RPT, RPT)])

    @pl.when(c == 0)
    def _():
        pltpu.sync_copy(e0_hbm.at[0, s], idx_s)
        pltpu.sync_copy(e0_hbm.at[1, s], idx_d)

    @pl.when(c == 1)
    def _():
        pltpu.sync_copy(e1_hbm.at[0, s], idx_s)
        pltpu.sync_copy(e1_hbm.at[1, s], idx_d)

    plsc.subcore_barrier()

    def run(t_hbm):
        def gather(k, b):
            return pltpu.make_async_copy(t_hbm.at[idx_s.at[k]], rows.at[b],
                                         sems[b])

        for b in range(NBUF):
            gather(b, b).start()

        def body(i, _):
            for b in range(NBUF):
                k = i * NBUF + b
                gather(k, b).wait()
                pltpu.sync_copy(rows.at[b], agg_sh.at[idx_d.at[k]], add=True)

                @pl.when(k + NBUF < NCHUNKC)
                def _():
                    gather(k + NBUF, b).start()
            return 0

        lax.fori_loop(0, NCHUNKC // NBUF, body, 0)

    @pl.when(c == 0)
    def _():
        run(t0_hbm)

    @pl.when(c == 1)
    def _():
        run(t1_hbm)

    plsc.subcore_barrier()

    row_slice = pl.ds(s * RPT, RPT)

    @pl.when(c == 0)
    def _():
        pltpu.sync_copy(agg_sh.at[row_slice], agg0_hbm.at[row_slice])

    @pl.when(c == 1)
    def _():
        pltpu.sync_copy(agg_sh.at[row_slice], agg1_hbm.at[row_slice])


_sc_scatter = pl.kernel(
    _scatter_body,
    out_type=(jax.ShapeDtypeStruct((NPAD, DH), jnp.float32),
              jax.ShapeDtypeStruct((NPAD, DH), jnp.float32)),
    mesh=_mesh,
    scratch_types=[
        pltpu.VMEM((NCHUNKC, KC), jnp.int32),
        pltpu.VMEM((NCHUNKC, KC), jnp.int32),
        pltpu.VMEM((NBUF, KC, DH), jnp.float32),
        pltpu.VMEM_SHARED((NPAD, DH), jnp.float32),
        pltpu.SemaphoreType.DMA,
        pltpu.SemaphoreType.DMA,
        pltpu.SemaphoreType.DMA,
        pltpu.SemaphoreType.DMA,
    ],
    compiler_params=pltpu.CompilerParams(use_tc_tiling_on_sc=False),
)


def _combine_body(a0a_ref, a0b_ref, a1a_ref, a1b_ref, d0_ref, d1_ref,
                  b0_ref, b1_ref, out_ref):
    s0 = lax.rsqrt(jnp.maximum(d0_ref[...], 1.0))
    s1 = lax.rsqrt(jnp.maximum(d1_ref[...], 1.0))
    b = b0_ref[...] + b1_ref[...]
    ha = a0a_ref[...] * s0 + a1a_ref[...] * s1 + b[:, :DH]
    hb = a0b_ref[...] * s0 + a1b_ref[...] * s1 + b[:, DH:]
    ss = (jnp.sum(ha * ha, axis=1, keepdims=True)
          + jnp.sum(hb * hb, axis=1, keepdims=True))
    inv = lax.rsqrt(jnp.maximum(ss, 1e-24))
    out_ref[:, :DH] = ha * inv
    out_ref[:, DH:] = hb * inv


def _tc_combine(a0a, a0b, a1a, a1b, d0, d1, b0, b1):
    blk = 1000
    grid = (N // blk,)
    half = pl.BlockSpec((blk, DH), lambda i: (i, 0))
    return pl.pallas_call(
        _combine_body,
        grid=grid,
        in_specs=[
            half, half, half, half,
            pl.BlockSpec((blk, 1), lambda i: (i, 0)),
            pl.BlockSpec((blk, 1), lambda i: (i, 0)),
            pl.BlockSpec((1, D), lambda i: (0, 0)),
            pl.BlockSpec((1, D), lambda i: (0, 0)),
        ],
        out_specs=pl.BlockSpec((blk, D), lambda i: (i, 0)),
        out_shape=jax.ShapeDtypeStruct((N, D), jnp.float32),
    )(a0a, a0b, a1a, a1b, d0, d1, b0, b1)


@jax.jit
def kernel(x, edge_index_r0, edge_index_r1, W_r0, b_r0, W_r1, b_r1):
    e0 = edge_index_r0.reshape(2, NS, NCHUNK, K)
    e1 = edge_index_r1.reshape(2, NS, NCHUNK, K)
    e0c = edge_index_r0.reshape(2, NS, NCHUNKC, KC)
    e1c = edge_index_r1.reshape(2, NS, NCHUNKC, KC)

    deg0, deg1 = _sc_degrees(e0, e1)
    dout0 = deg0[0, :N, None]
    din0 = deg0[1, :N, None]
    dout1 = deg1[0, :N, None]
    din1 = deg1[1, :N, None]

    t0a, t0b, t1a, t1b = _tc_matmul(x, W_r0, W_r1, dout0, dout1)

    zeros = jnp.zeros((RPT, DH), jnp.float32)
    a0a, a1a = _sc_scatter(t0a, t1a, e0c, e1c, zeros)
    a0b, a1b = _sc_scatter(t0b, t1b, e0c, e1c, zeros)

    return _tc_combine(a0a[:N], a0b[:N], a1a[:N], a1b[:N], din0, din1,
                       b_r0[None, :], b_r1[None, :])


# speedup vs baseline: 13.4816x; 1.0014x over previous
import jax
import jax.numpy as jnp
from jax import lax
from jax.experimental import pallas as pl
from jax.experimental.pallas import tpu as pltpu
from jax.experimental.pallas import tpu_sc as plsc

N = 10000
E = 320000
D = 128
DH = D // 2

NC = 2
NS = 16
K = 80
NCHUNK = E // (NS * K)
KC = 125
NCHUNKC = E // (NS * KC)
NBUF = 4
NPAD = 10240
RPT = NPAD // NS
HSLICE = NPAD // NS

_mesh = plsc.VectorSubcoreMesh(core_axis_name="c", subcore_axis_name="s")


def _degrees_body(e0_hbm, e1_hbm, out0_hbm, out1_hbm,
                  hist_s, hist_d, idx_s, idx_d, tmp, acc_s, acc_d, sh_s, sh_d):
    c = lax.axis_index("c")
    s = lax.axis_index("s")
    zero16 = jnp.zeros((16,), jnp.float32)

    def zero_body(i, _):
        hist_s[pl.ds(i * 16, 16)] = zero16
        hist_d[pl.ds(i * 16, 16)] = zero16
        return 0

    lax.fori_loop(0, NPAD // 16, zero_body, 0)

    @pl.when(c == 0)
    def _():
        pltpu.sync_copy(e0_hbm.at[0, s], idx_s)
        pltpu.sync_copy(e0_hbm.at[1, s], idx_d)

    @pl.when(c == 1)
    def _():
        pltpu.sync_copy(e1_hbm.at[0, s], idx_s)
        pltpu.sync_copy(e1_hbm.at[1, s], idx_d)

    one16 = jnp.full((16,), 1.0, jnp.float32)

    def acc_body(r, _):
        for j in range(K // 16):
            plsc.addupdate_scatter(hist_s, [idx_s[r, pl.ds(j * 16, 16)]], one16)
            plsc.addupdate_scatter(hist_d, [idx_d[r, pl.ds(j * 16, 16)]], one16)
        return 0

    lax.fori_loop(0, NCHUNK, acc_body, 0)

    pltpu.sync_copy(hist_s, sh_s.at[s])
    pltpu.sync_copy(hist_d, sh_d.at[s])
    plsc.subcore_barrier()

    col = pl.ds(s * HSLICE, HSLICE)

    def reduce(sh, acc):
        for t in range(NS):
            pltpu.sync_copy(sh.at[t, col], tmp.at[t])

        def red_body(j, _):
            v = tmp[0, pl.ds(j * 16, 16)]
            for t in range(1, NS):
                v = v + tmp[t, pl.ds(j * 16, 16)]
            acc[pl.ds(j * 16, 16)] = v
            return 0

        lax.fori_loop(0, HSLICE // 16, red_body, 0)

    reduce(sh_s, acc_s)
    reduce(sh_d, acc_d)

    @pl.when(c == 0)
    def _():
        pltpu.sync_copy(acc_s, out0_hbm.at[0, col])
        pltpu.sync_copy(acc_d, out0_hbm.at[1, col])

    @pl.when(c == 1)
    def _():
        pltpu.sync_copy(acc_s, out1_hbm.at[0, col])
        pltpu.sync_copy(acc_d, out1_hbm.at[1, col])


_sc_degrees = pl.kernel(
    _degrees_body,
    out_type=(jax.ShapeDtypeStruct((2, NPAD), jnp.float32),
              jax.ShapeDtypeStruct((2, NPAD), jnp.float32)),
    mesh=_mesh,
    scratch_types=[
        pltpu.VMEM((NPAD,), jnp.float32),
        pltpu.VMEM((NPAD,), jnp.float32),
        pltpu.VMEM((NCHUNK, K), jnp.int32),
        pltpu.VMEM((NCHUNK, K), jnp.int32),
        pltpu.VMEM((NS, HSLICE), jnp.float32),
        pltpu.VMEM((HSLICE,), jnp.float32),
        pltpu.VMEM((HSLICE,), jnp.float32),
        pltpu.VMEM_SHARED((NS, NPAD), jnp.float32),
        pltpu.VMEM_SHARED((NS, NPAD), jnp.float32),
    ],
    compiler_params=pltpu.CompilerParams(needs_layout_passes=False),
)


def _matmul_body(x_ref, w0_ref, w1_ref, d0_ref, d1_ref,
                 t0a_ref, t0b_ref, t1a_ref, t1b_ref):
    x = x_ref[...]
    s0 = lax.rsqrt(jnp.maximum(d0_ref[...], 1.0))
    s1 = lax.rsqrt(jnp.maximum(d1_ref[...], 1.0))
    t0 = jnp.dot(x * s0, w0_ref[...], preferred_element_type=jnp.float32)
    t1 = jnp.dot(x * s1, w1_ref[...], preferred_element_type=jnp.float32)
    t0a_ref[...] = t0[:, :DH]
    t0b_ref[...] = t0[:, DH:]
    t1a_ref[...] = t1[:, :DH]
    t1b_ref[...] = t1[:, DH:]


def _tc_matmul(x, w0, w1, d0, d1):
    blk = 1000
    grid = (N // blk,)
    half = pl.BlockSpec((blk, DH), lambda i: (i, 0))
    return pl.pallas_call(
        _matmul_body,
        grid=grid,
        in_specs=[
            pl.BlockSpec((blk, D), lambda i: (i, 0)),
            pl.BlockSpec((D, D), lambda i: (0, 0)),
            pl.BlockSpec((D, D), lambda i: (0, 0)),
            pl.BlockSpec((blk, 1), lambda i: (i, 0)),
            pl.BlockSpec((blk, 1), lambda i: (i, 0)),
        ],
        out_specs=[half, half, half, half],
        out_shape=[jax.ShapeDtypeStruct((N, DH), jnp.float32)] * 4,
    )(x, w0, w1, d0, d1)


def _scatter_body(t0_hbm, t1_hbm, e0_hbm, e1_hbm, z_hbm, agg0_hbm, agg1_hbm,
                  idx_s, idx_d, rows, agg_sh, sem0, sem1, sem2, sem3):
    c = lax.axis_index("c")
    s = lax.axis_index("s")
    sems = (sem0, sem1, sem2, sem3)

    pltpu.sync_copy(z_hbm, agg_sh.at[pl.ds(s * RPT, RPT)])

    @pl.when(c == 0)
    def _():
        pltpu.sync_copy(e0_hbm.at[0, s], idx_s)
        pltpu.sync_copy(e0_hbm.at[1, s], idx_d)

    @pl.when(c == 1)
    def _():
        pltpu.sync_copy(e1_hbm.at[0, s], idx_s)
        pltpu.sync_copy(e1_hbm.at[1, s], idx_d)

    plsc.subcore_barrier()

    def run(t_hbm):
        def gather(k, b):
            return pltpu.make_async_copy(t_hbm.at[idx_s.at[k]], rows.at[b],
                                         sems[b])

        for b in range(NBUF):
            gather(b, b).start()

        def body(i, _):
            for b in range(NBUF):
                k = i * NBUF + b
                gather(k, b).wait()
                pltpu.sync_copy(rows.at[b], agg_sh.at[idx_d.at[k]], add=True)

                @pl.when(k + NBUF < NCHUNKC)
                def _():
                    gather(k + NBUF, b).start()
            return 0

        lax.fori_loop(0, NCHUNKC // NBUF, body, 0)

    @pl.when(c == 0)
    def _():
        run(t0_hbm)

    @pl.when(c == 1)
    def _():
        run(t1_hbm)

    plsc.subcore_barrier()

    row_slice = pl.ds(s * RPT, RPT)

    @pl.when(c == 0)
    def _():
        pltpu.sync_copy(agg_sh.at[row_slice], agg0_hbm.at[row_slice])

    @pl.when(c == 1)
    def _():
        pltpu.sync_copy(agg_sh.at[row_slice], agg1_hbm.at[row_slice])


_sc_scatter = pl.kernel(
    _scatter_body,
    out_type=(jax.ShapeDtypeStruct((NPAD, DH), jnp.float32),
              jax.ShapeDtypeStruct((NPAD, DH), jnp.float32)),
    mesh=_mesh,
    scratch_types=[
        pltpu.VMEM((NCHUNKC, KC), jnp.int32),
        pltpu.VMEM((NCHUNKC, KC), jnp.int32),
        pltpu.VMEM((NBUF, KC, DH), jnp.float32),
        pltpu.VMEM_SHARED((NPAD, DH), jnp.float32),
        pltpu.SemaphoreType.DMA,
        pltpu.SemaphoreType.DMA,
        pltpu.SemaphoreType.DMA,
        pltpu.SemaphoreType.DMA,
    ],
    compiler_params=pltpu.CompilerParams(use_tc_tiling_on_sc=False),
)


def _combine_body(a0a_ref, a0b_ref, a1a_ref, a1b_ref, d0_ref, d1_ref,
                  b0_ref, b1_ref, out_ref):
    s0 = lax.rsqrt(jnp.maximum(d0_ref[...], 1.0))
    s1 = lax.rsqrt(jnp.maximum(d1_ref[...], 1.0))
    b = b0_ref[...] + b1_ref[...]
    ha = a0a_ref[...] * s0 + a1a_ref[...] * s1 + b[:, :DH]
    hb = a0b_ref[...] * s0 + a1b_ref[...] * s1 + b[:, DH:]
    ss = (jnp.sum(ha * ha, axis=1, keepdims=True)
          + jnp.sum(hb * hb, axis=1, keepdims=True))
    inv = lax.rsqrt(jnp.maximum(ss, 1e-24))
    out_ref[:, :DH] = ha * inv
    out_ref[:, DH:] = hb * inv


def _tc_combine(a0a, a0b, a1a, a1b, d0, d1, b0, b1):
    blk = 1000
    grid = (N // blk,)
    half = pl.BlockSpec((blk, DH), lambda i: (i, 0))
    return pl.pallas_call(
        _combine_body,
        grid=grid,
        in_specs=[
            half, half, half, half,
            pl.BlockSpec((blk, 1), lambda i: (i, 0)),
            pl.BlockSpec((blk, 1), lambda i: (i, 0)),
            pl.BlockSpec((1, D), lambda i: (0, 0)),
            pl.BlockSpec((1, D), lambda i: (0, 0)),
        ],
        out_specs=pl.BlockSpec((blk, D), lambda i: (i, 0)),
        out_shape=jax.ShapeDtypeStruct((N, D), jnp.float32),
    )(a0a, a0b, a1a, a1b, d0, d1, b0, b1)


@jax.jit
def kernel(x, edge_index_r0, edge_index_r1, W_r0, b_r0, W_r1, b_r1):
    e0 = edge_index_r0.reshape(2, NS, NCHUNK, K)
    e1 = edge_index_r1.reshape(2, NS, NCHUNK, K)
    e0c = edge_index_r0.reshape(2, NS, NCHUNKC, KC)
    e1c = edge_index_r1.reshape(2, NS, NCHUNKC, KC)

    deg0, deg1 = _sc_degrees(e0, e1)
    dout0 = deg0[0, :N, None]
    din0 = deg0[1, :N, None]
    dout1 = deg1[0, :N, None]
    din1 = deg1[1, :N, None]

    t0a, t0b, t1a, t1b = _tc_matmul(x, W_r0, W_r1, dout0, dout1)

    zeros = jnp.zeros((RPT, DH), jnp.float32)
    a0a, a1a = _sc_scatter(t0a, t1a, e0c, e1c, zeros)
    a0b, a1b = _sc_scatter(t0b, t1b, e0c, e1c, zeros)

    return _tc_combine(a0a[:N], a0b[:N], a1a[:N], a1b[:N], din0, din1,
                       b_r0[None, :], b_r1[None, :])


# speedup vs baseline: 13.6440x; 1.0120x over previous
import jax
import jax.numpy as jnp
from jax import lax
from jax.experimental import pallas as pl
from jax.experimental.pallas import tpu as pltpu
from jax.experimental.pallas import tpu_sc as plsc

N = 10000
E = 320000
D = 128
DH = D // 2

NC = 2
NS = 16
K = 80
NCHUNK = E // (NS * K)
KC = 125
NCHUNKC = E // (NS * KC)
NBUF = 4
NPAD = 10240
RPT = NPAD // NS
HSLICE = NPAD // NS

_mesh = plsc.VectorSubcoreMesh(core_axis_name="c", subcore_axis_name="s")


def _degrees_body(e0_hbm, e1_hbm, out0_hbm, out1_hbm,
                  hist_s, hist_d, idx_s, idx_d, tmp, acc_s, acc_d, sh_s, sh_d):
    c = lax.axis_index("c")
    s = lax.axis_index("s")
    zero16 = jnp.zeros((16,), jnp.float32)

    def zero_body(i, _):
        hist_s[pl.ds(i * 16, 16)] = zero16
        hist_d[pl.ds(i * 16, 16)] = zero16
        return 0

    lax.fori_loop(0, NPAD // 16, zero_body, 0)

    @pl.when(c == 0)
    def _():
        pltpu.sync_copy(e0_hbm.at[0, s], idx_s)
        pltpu.sync_copy(e0_hbm.at[1, s], idx_d)

    @pl.when(c == 1)
    def _():
        pltpu.sync_copy(e1_hbm.at[0, s], idx_s)
        pltpu.sync_copy(e1_hbm.at[1, s], idx_d)

    one16 = jnp.full((16,), 1.0, jnp.float32)

    def acc_body(r, _):
        for j in range(K // 16):
            plsc.addupdate_scatter(hist_s, [idx_s[r, pl.ds(j * 16, 16)]], one16)
            plsc.addupdate_scatter(hist_d, [idx_d[r, pl.ds(j * 16, 16)]], one16)
        return 0

    lax.fori_loop(0, NCHUNK, acc_body, 0)

    pltpu.sync_copy(hist_s, sh_s.at[s])
    pltpu.sync_copy(hist_d, sh_d.at[s])
    plsc.subcore_barrier()

    col = pl.ds(s * HSLICE, HSLICE)

    def reduce(sh, acc):
        for t in range(NS):
            pltpu.sync_copy(sh.at[t, col], tmp.at[t])

        def red_body(j, _):
            v = tmp[0, pl.ds(j * 16, 16)]
            for t in range(1, NS):
                v = v + tmp[t, pl.ds(j * 16, 16)]
            acc[pl.ds(j * 16, 16)] = v
            return 0

        lax.fori_loop(0, HSLICE // 16, red_body, 0)

    reduce(sh_s, acc_s)
    reduce(sh_d, acc_d)

    @pl.when(c == 0)
    def _():
        pltpu.sync_copy(acc_s, out0_hbm.at[0, col])
        pltpu.sync_copy(acc_d, out0_hbm.at[1, col])

    @pl.when(c == 1)
    def _():
        pltpu.sync_copy(acc_s, out1_hbm.at[0, col])
        pltpu.sync_copy(acc_d, out1_hbm.at[1, col])


_sc_degrees = pl.kernel(
    _degrees_body,
    out_type=(jax.ShapeDtypeStruct((2, NPAD), jnp.float32),
              jax.ShapeDtypeStruct((2, NPAD), jnp.float32)),
    mesh=_mesh,
    scratch_types=[
        pltpu.VMEM((NPAD,), jnp.float32),
        pltpu.VMEM((NPAD,), jnp.float32),
        pltpu.VMEM((NCHUNK, K), jnp.int32),
        pltpu.VMEM((NCHUNK, K), jnp.int32),
        pltpu.VMEM((NS, HSLICE), jnp.float32),
        pltpu.VMEM((HSLICE,), jnp.float32),
        pltpu.VMEM((HSLICE,), jnp.float32),
        pltpu.VMEM_SHARED((NS, NPAD), jnp.float32),
        pltpu.VMEM_SHARED((NS, NPAD), jnp.float32),
    ],
    compiler_params=pltpu.CompilerParams(needs_layout_passes=False),
)


def _matmul_body(x_ref, w0_ref, w1_ref, d0_ref, d1_ref,
                 t0a_ref, t0b_ref, t1a_ref, t1b_ref):
    x = x_ref[...]
    s0 = lax.rsqrt(jnp.maximum(d0_ref[...], 1.0))
    s1 = lax.rsqrt(jnp.maximum(d1_ref[...], 1.0))
    t0 = jnp.dot(x * s0, w0_ref[...], preferred_element_type=jnp.float32)
    t1 = jnp.dot(x * s1, w1_ref[...], preferred_element_type=jnp.float32)
    t0a_ref[...] = t0[:, :DH]
    t0b_ref[...] = t0[:, DH:]
    t1a_ref[...] = t1[:, :DH]
    t1b_ref[...] = t1[:, DH:]


def _tc_matmul(x, w0, w1, d0, d1):
    blk = 1000
    grid = (N // blk,)
    half = pl.BlockSpec((blk, DH), lambda i: (i, 0))
    return pl.pallas_call(
        _matmul_body,
        grid=grid,
        in_specs=[
            pl.BlockSpec((blk, D), lambda i: (i, 0)),
            pl.BlockSpec((D, D), lambda i: (0, 0)),
            pl.BlockSpec((D, D), lambda i: (0, 0)),
            pl.BlockSpec((blk, 1), lambda i: (i, 0)),
            pl.BlockSpec((blk, 1), lambda i: (i, 0)),
        ],
        out_specs=[half, half, half, half],
        out_shape=[jax.ShapeDtypeStruct((N, DH), jnp.float32)] * 4,
    )(x, w0, w1, d0, d1)


def _scatter_body(t0_hbm, t1_hbm, e0_hbm, e1_hbm, z_hbm, agg0_hbm, agg1_hbm,
                  idx_s, idx_d, rows, agg_sh, sem0, sem1, sem2, sem3):
    c = lax.axis_index("c")
    s = lax.axis_index("s")
    sems = (sem0, sem1, sem2, sem3)

    pltpu.sync_copy(z_hbm, agg_sh.at[pl.ds(s * RPT, RPT)])

    @pl.when(c == 0)
    def _():
        pltpu.sync_copy(e0_hbm.at[0, s], idx_s)
        pltpu.sync_copy(e0_hbm.at[1, s], idx_d)

    @pl.when(c == 1)
    def _():
        pltpu.sync_copy(e1_hbm.at[0, s], idx_s)
        pltpu.sync_copy(e1_hbm.at[1, s], idx_d)

    plsc.subcore_barrier()

    def run(t_hbm):
        def gather(k, b):
            return pltpu.make_async_copy(t_hbm.at[idx_s.at[k]], rows.at[b],
                                         sems[b])

        for b in range(NBUF):
            gather(b, b).start()

        def body(i, _):
            for b in range(NBUF):
                k = i * NBUF + b
                gather(k, b).wait()
                pltpu.sync_copy(rows.at[b], agg_sh.at[idx_d.at[k]], add=True)

                @pl.when(k + NBUF < NCHUNKC)
                def _():
                    gather(k + NBUF, b).start()
            return 0

        lax.fori_loop(0, NCHUNKC // NBUF, body, 0)

    @pl.when(c == 0)
    def _():
        run(t0_hbm)

    @pl.when(c == 1)
    def _():
        run(t1_hbm)

    plsc.subcore_barrier()

    row_slice = pl.ds(s * RPT, RPT)

    @pl.when(c == 0)
    def _():
        pltpu.sync_copy(agg_sh.at[row_slice], agg0_hbm.at[row_slice])

    @pl.when(c == 1)
    def _():
        pltpu.sync_copy(agg_sh.at[row_slice], agg1_hbm.at[row_slice])


_sc_scatter = pl.kernel(
    _scatter_body,
    out_type=(jax.ShapeDtypeStruct((NPAD, DH), jnp.float32),
              jax.ShapeDtypeStruct((NPAD, DH), jnp.float32)),
    mesh=_mesh,
    scratch_types=[
        pltpu.VMEM((NCHUNKC, KC), jnp.int32),
        pltpu.VMEM((NCHUNKC, KC), jnp.int32),
        pltpu.VMEM((NBUF, KC, DH), jnp.float32),
        pltpu.VMEM_SHARED((NPAD, DH), jnp.float32),
        pltpu.SemaphoreType.DMA,
        pltpu.SemaphoreType.DMA,
        pltpu.SemaphoreType.DMA,
        pltpu.SemaphoreType.DMA,
    ],
    compiler_params=pltpu.CompilerParams(use_tc_tiling_on_sc=False),
)


def _combine_body(a0a_ref, a0b_ref, a1a_ref, a1b_ref, d0_ref, d1_ref,
                  b0_ref, b1_ref, out_ref):
    s0 = lax.rsqrt(jnp.maximum(d0_ref[...], 1.0))
    s1 = lax.rsqrt(jnp.maximum(d1_ref[...], 1.0))
    b = b0_ref[...] + b1_ref[...]
    ha = a0a_ref[...] * s0 + a1a_ref[...] * s1 + b[:, :DH]
    hb = a0b_ref[...] * s0 + a1b_ref[...] * s1 + b[:, DH:]
    ss = (jnp.sum(ha * ha, axis=1, keepdims=True)
          + jnp.sum(hb * hb, axis=1, keepdims=True))
    inv = lax.rsqrt(jnp.maximum(ss, 1e-24))
    out_ref[:, :DH] = ha * inv
    out_ref[:, DH:] = hb * inv


def _tc_combine(a0a, a0b, a1a, a1b, d0, d1, b0, b1):
    blk = 1000
    grid = (N // blk,)
    half = pl.BlockSpec((blk, DH), lambda i: (i, 0))
    return pl.pallas_call(
        _combine_body,
        grid=grid,
        in_specs=[
            half, half, half, half,
            pl.BlockSpec((blk, 1), lambda i: (i, 0)),
            pl.BlockSpec((blk, 1), lambda i: (i, 0)),
            pl.BlockSpec((1, D), lambda i: (0, 0)),
            pl.BlockSpec((1, D), lambda i: (0, 0)),
        ],
        out_specs=pl.BlockSpec((blk, D), lambda i: (i, 0)),
        out_shape=jax.ShapeDtypeStruct((N, D), jnp.float32),
    )(a0a, a0b, a1a, a1b, d0, d1, b0, b1)


@jax.jit
def kernel(x, edge_index_r0, edge_index_r1, W_r0, b_r0, W_r1, b_r1):
    e0 = edge_index_r0.reshape(2, NS, NCHUNK, K)
    e1 = edge_index_r1.reshape(2, NS, NCHUNK, K)
    e0c = edge_index_r0.reshape(2, NS, NCHUNKC, KC)
    e1c = edge_index_r1.reshape(2, NS, NCHUNKC, KC)

    deg0, deg1 = _sc_degrees(e0, e1)
    dout0 = deg0[0, :N, None]
    din0 = deg0[1, :N, None]
    dout1 = deg1[0, :N, None]
    din1 = deg1[1, :N, None]

    t0a, t0b, t1a, t1b = _tc_matmul(x, W_r0, W_r1, dout0, dout1)

    zeros = jnp.zeros((RPT, DH), jnp.float32)
    a0a, a1a = _sc_scatter(t0a, t1a, e0c, e1c, zeros)
    a0b, a1b = _sc_scatter(t0b, t1b, e0c, e1c, zeros)

    return _tc_combine(a0a, a0b, a1a, a1b, din0, din1,
                       b_r0[None, :], b_r1[None, :])
